# Initial kernel scaffold; baseline (speedup 1.0000x reference)
import jax
import jax.numpy as jnp
from jax.experimental import pallas as pl


def kernel(x, W, b):
    raise NotImplementedError("write your pallas kernel here")


# speedup vs baseline: 16.0443x; 16.0443x over previous
import functools

import jax
import jax.numpy as jnp
from jax import lax
from jax.experimental import pallas as pl
from jax.experimental.pallas import tpu as pltpu
from jax.experimental.pallas import tpu_sc as plsc

B, C, N, K = 4, 256, 4096, 16
C_OUT = 256
R = 256


def _tc_body(xt_ref, xc_ref, wu_ref, wv_ref, b2_ref, nn_ref, ut_ref, vt_ref):
    xtb = xt_ref[0]
    xca = xc_ref[0]
    s = jax.lax.dot_general(
        xtb, xca, (((1,), (0,)), ((), ())),
        preferred_element_type=jnp.float32)
    colsq = jnp.sum(xca * xca, axis=0, keepdims=True)
    d = colsq - 2.0 * s

    iota = jax.lax.broadcasted_iota(jnp.int32, (R, N), 1)
    big = jnp.int32(N)
    inf = jnp.float32(jnp.inf)
    for t in range(K):
        m = jnp.min(d, axis=1, keepdims=True)
        cand = jnp.where(d <= m, iota, big)
        idx = jnp.min(cand, axis=1, keepdims=True)
        nn_ref[0, :, pl.ds(t, 1)] = idx
        d = jnp.where(cand == idx, inf, d)

    ut_ref[0] = jax.lax.dot_general(
        xtb, wu_ref[...], (((1,), (0,)), ((), ())),
        precision=jax.lax.Precision.HIGHEST,
        preferred_element_type=jnp.float32) + b2_ref[...]
    vt_ref[0] = jax.lax.dot_general(
        xtb, wv_ref[...], (((1,), (0,)), ((), ())),
        precision=jax.lax.Precision.HIGHEST,
        preferred_element_type=jnp.float32)


def _tc_stage(xt, xc, wu, wv, b2):
    grid = (B, N // R)
    return pl.pallas_call(
        _tc_body,
        grid=grid,
        in_specs=[
            pl.BlockSpec((1, R, C), lambda b, i: (b, i, 0)),
            pl.BlockSpec((1, C, N), lambda b, i: (b, 0, 0)),
            pl.BlockSpec((C, C_OUT), lambda b, i: (0, 0)),
            pl.BlockSpec((C, C_OUT), lambda b, i: (0, 0)),
            pl.BlockSpec((1, C_OUT), lambda b, i: (0, 0)),
        ],
        out_specs=[
            pl.BlockSpec((1, R, K), lambda b, i: (b, i, 0)),
            pl.BlockSpec((1, R, C_OUT), lambda b, i: (b, i, 0)),
            pl.BlockSpec((1, R, C_OUT), lambda b, i: (b, i, 0)),
        ],
        out_shape=[
            jax.ShapeDtypeStruct((B, N, K), jnp.int32),
            jax.ShapeDtypeStruct((B, N, C_OUT), jnp.float32),
            jax.ShapeDtypeStruct((B, N, C_OUT), jnp.float32),
        ],
    )(xt, xc, wu, wv, b2)


_PTS = 8
_L = 16


def _sc_gather_max():
    info = plsc.get_sparse_core_info()
    nc, ns = info.num_cores, info.num_subcores
    nw = nc * ns
    bn = B * N
    per_w = bn // nw
    groups = per_w // _PTS
    mesh = plsc.VectorSubcoreMesh(core_axis_name="c", subcore_axis_name="s")

    @functools.partial(
        pl.kernel,
        mesh=mesh,
        out_type=jax.ShapeDtypeStruct((bn, C_OUT), jnp.float32),
        scratch_types=[
            pltpu.VMEM((_PTS * K,), jnp.int32),
            pltpu.VMEM((_PTS * K,), jnp.int32),
            pltpu.VMEM((_PTS * K, C_OUT), jnp.float32),
            pltpu.VMEM((_PTS, C_OUT), jnp.float32),
            pltpu.VMEM((_PTS, C_OUT), jnp.float32),
            pltpu.SemaphoreType.DMA,
        ],
    )
    def k(vt_hbm, ut_hbm, nn_hbm, out_hbm, idx_v, idx2_v, rows_v, u_v, o_v, sem):
        wid = lax.axis_index("s") * nc + lax.axis_index("c")
        base = wid * per_w
        boff = (base // N) * N

        def group(g, _):
            p0 = base + g * _PTS
            pltpu.sync_copy(nn_hbm.at[pl.ds(p0 * K, _PTS * K)], idx_v)
            for c in range(_PTS * K // _L):
                idx2_v[pl.ds(c * _L, _L)] = idx_v[pl.ds(c * _L, _L)] + boff
            pltpu.async_copy(vt_hbm.at[idx2_v], rows_v, sem).wait()
            pltpu.sync_copy(ut_hbm.at[pl.ds(p0, _PTS)], u_v)

            def point(j, _):
                for c in range(C_OUT // _L):
                    sl = pl.ds(c * _L, _L)
                    acc = rows_v[j * K, sl]
                    for r in range(1, K):
                        acc = jnp.maximum(acc, rows_v[j * K + r, sl])
                    o_v[j, sl] = jnp.maximum(acc + u_v[j, sl], 0.0)
                return 0

            lax.fori_loop(0, _PTS, point, 0)
            pltpu.sync_copy(o_v, out_hbm.at[pl.ds(p0, _PTS)])
            return 0

        lax.fori_loop(0, groups, group, 0)

    return k


def kernel(x, W, b):
    xc = x[..., 0]
    xt = jnp.transpose(xc, (0, 2, 1))
    w1 = W[:, :C]
    w2 = W[:, C:]
    wu = jnp.transpose(w1 - w2)
    wv = jnp.transpose(w2)
    b2 = b[None, :]

    nn, ut, vt = _tc_stage(xt, xc, wu, wv, b2)

    g = _sc_gather_max()(
        vt.reshape(B * N, C_OUT),
        ut.reshape(B * N, C_OUT),
        nn.reshape(B * N * K),
    )
    return jnp.transpose(g.reshape(B, N, C_OUT), (0, 2, 1))[..., None]


# speedup vs baseline: 21.8709x; 1.3632x over previous
import functools

import jax
import jax.numpy as jnp
from jax import lax
from jax.experimental import pallas as pl
from jax.experimental.pallas import tpu as pltpu
from jax.experimental.pallas import tpu_sc as plsc

B, C, N, K = 4, 256, 4096, 16
C_OUT = 256
R = 256


def _tc_body(xt_ref, xc_ref, wu_ref, wv_ref, b2_ref, nn_ref, ut_ref, vt_ref):
    row0 = pl.program_id(0) * R
    xtb = xt_ref[...]
    xca = xc_ref[...]
    s = jax.lax.dot_general(
        xtb, xca, (((1,), (0,)), ((), ())),
        preferred_element_type=jnp.float32)
    colsq = jnp.sum(xca * xca, axis=0, keepdims=True)
    d = colsq - 2.0 * s

    iota = jax.lax.broadcasted_iota(jnp.int32, (R, N), 1).astype(jnp.float32)
    big = jnp.float32(N)
    inf = jnp.float32(jnp.inf)

    row_ids = row0 + jax.lax.broadcasted_iota(jnp.int32, (R, 1), 0)
    nn_ref[:, pl.ds(0, 1)] = row_ids
    d = jnp.where(iota == row_ids.astype(jnp.float32), inf, d)

    for t in range(1, K):
        m = jnp.min(d, axis=1, keepdims=True)
        cand = jnp.where(d <= m, iota, big)
        idx = jnp.min(cand, axis=1, keepdims=True)
        nn_ref[:, pl.ds(t, 1)] = idx.astype(jnp.int32)
        d = jnp.where(cand == idx, inf, d)

    ut_ref[...] = jax.lax.dot_general(
        xtb, wu_ref[...], (((1,), (0,)), ((), ())),
        precision=jax.lax.Precision.HIGHEST,
        preferred_element_type=jnp.float32) + b2_ref[...]
    vt_ref[...] = jax.lax.dot_general(
        xtb, wv_ref[...], (((1,), (0,)), ((), ())),
        precision=jax.lax.Precision.HIGHEST,
        preferred_element_type=jnp.float32)


@functools.cache
def _tc_stage():
    return pl.pallas_call(
        _tc_body,
        grid=(N // R,),
        in_specs=[
            pl.BlockSpec((R, C), lambda i: (i, 0)),
            pl.BlockSpec((C, N), lambda i: (0, 0)),
            pl.BlockSpec((C, C_OUT), lambda i: (0, 0)),
            pl.BlockSpec((C, C_OUT), lambda i: (0, 0)),
            pl.BlockSpec((1, C_OUT), lambda i: (0, 0)),
        ],
        out_specs=[
            pl.BlockSpec((R, K), lambda i: (i, 0)),
            pl.BlockSpec((R, C_OUT), lambda i: (i, 0)),
            pl.BlockSpec((R, C_OUT), lambda i: (i, 0)),
        ],
        out_shape=[
            jax.ShapeDtypeStruct((N, K), jnp.int32),
            jax.ShapeDtypeStruct((N, C_OUT), jnp.float32),
            jax.ShapeDtypeStruct((N, C_OUT), jnp.float32),
        ],
    )


_PTS = 8
_L = 16


@functools.cache
def _sc_gather_max():
    info = plsc.get_sparse_core_info()
    nc, ns = info.num_cores, info.num_subcores
    nw = nc * ns
    per_w = N // nw
    groups = per_w // _PTS
    mesh = plsc.VectorSubcoreMesh(core_axis_name="c", subcore_axis_name="s")

    @functools.partial(
        pl.kernel,
        mesh=mesh,
        out_type=jax.ShapeDtypeStruct((N, C_OUT), jnp.float32),
        scratch_types=[
            pltpu.VMEM((_PTS * K,), jnp.int32),
            pltpu.VMEM((_PTS * K, C_OUT), jnp.float32),
            pltpu.VMEM((_PTS, C_OUT), jnp.float32),
            pltpu.VMEM((_PTS, C_OUT), jnp.float32),
            pltpu.SemaphoreType.DMA,
        ],
    )
    def k(vt_hbm, ut_hbm, nn_hbm, out_hbm, idx_v, rows_v, u_v, o_v, sem):
        wid = lax.axis_index("s") * nc + lax.axis_index("c")
        base = wid * per_w

        def group(g, _):
            p0 = base + g * _PTS
            pltpu.sync_copy(nn_hbm.at[pl.ds(p0 * K, _PTS * K)], idx_v)
            pltpu.async_copy(vt_hbm.at[idx_v], rows_v, sem).wait()
            pltpu.sync_copy(ut_hbm.at[pl.ds(p0, _PTS)], u_v)

            def point(j, _):
                for c in range(C_OUT // _L):
                    sl = pl.ds(c * _L, _L)
                    acc = rows_v[j * K, sl]
                    for r in range(1, K):
                        acc = jnp.maximum(acc, rows_v[j * K + r, sl])
                    o_v[j, sl] = jnp.maximum(acc + u_v[j, sl], 0.0)
                return 0

            lax.fori_loop(0, _PTS, point, 0)
            pltpu.sync_copy(o_v, out_hbm.at[pl.ds(p0, _PTS)])
            return 0

        lax.fori_loop(0, groups, group, 0)

    return k


def kernel(x, W, b):
    xc = x[..., 0]
    xt = jnp.transpose(xc, (0, 2, 1))
    w1 = W[:, :C]
    w2 = W[:, C:]
    wu = jnp.transpose(w1 - w2)
    wv = jnp.transpose(w2)
    b2 = b[None, :]

    tc = _tc_stage()
    sc = _sc_gather_max()
    outs = []
    for bb in range(B):
        nn, ut, vt = tc(xt[bb], xc[bb], wu, wv, b2)
        outs.append(sc(vt, ut, nn.reshape(N * K)))
    g = jnp.stack(outs)
    return jnp.transpose(g, (0, 2, 1))[..., None]


# speedup vs baseline: 32.3707x; 1.4801x over previous
import functools

import jax
import jax.numpy as jnp
from jax import lax
from jax.experimental import pallas as pl
from jax.experimental.pallas import tpu as pltpu
from jax.experimental.pallas import tpu_sc as plsc

B, C, N, K = 4, 256, 4096, 16
C_OUT = 256
R = 256


def _tc_body(xt_ref, xc_ref, wu_ref, wv_ref, b2_ref, nn_ref, ut_ref, vt_ref):
    row0 = pl.program_id(0) * R
    xtb = xt_ref[...]
    xca = xc_ref[...]
    s = jax.lax.dot_general(
        xtb, xca, (((1,), (0,)), ((), ())),
        preferred_element_type=jnp.float32)
    colsq = jnp.sum(xca * xca, axis=0, keepdims=True)
    d = colsq - 2.0 * s

    iota = jax.lax.broadcasted_iota(jnp.int32, (R, N), 1)
    inf = jnp.float32(jnp.inf)

    row_ids = row0 + jax.lax.broadcasted_iota(jnp.int32, (R, 1), 0)
    nn_ref[:, pl.ds(0, 1)] = row_ids
    d = jnp.where(iota == row_ids, inf, d)

    nch = N // 128
    Dm = jnp.concatenate(
        [jnp.min(d[:, c * 128:(c + 1) * 128], axis=1, keepdims=True)
         for c in range(nch)], axis=1)
    iota_ch = jax.lax.broadcasted_iota(jnp.int32, (R, nch), 1)
    iota_l = jax.lax.broadcasted_iota(jnp.int32, (R, 128), 1)

    for t in range(1, K):
        m = jnp.min(Dm, axis=1, keepdims=True)
        cstar = jnp.argmin(Dm, axis=1).astype(jnp.int32)[:, None]
        g = d[:, 0:128]
        for c in range(1, nch):
            g = jnp.where(cstar == c, d[:, c * 128:(c + 1) * 128], g)
        g = jnp.where(g < m, inf, g)
        li = jnp.argmin(g, axis=1).astype(jnp.int32)[:, None]
        nn_ref[:, pl.ds(t, 1)] = cstar * 128 + li
        g = jnp.where(iota_l == li, inf, g)
        m2 = jnp.min(g, axis=1, keepdims=True)
        Dm = jnp.where(iota_ch == cstar, m2, Dm)

    ut_ref[...] = jax.lax.dot_general(
        xtb, wu_ref[...], (((1,), (0,)), ((), ())),
        precision=jax.lax.Precision.HIGHEST,
        preferred_element_type=jnp.float32) + b2_ref[...]
    vt_ref[...] = jax.lax.dot_general(
        xtb, wv_ref[...], (((1,), (0,)), ((), ())),
        precision=jax.lax.Precision.HIGHEST,
        preferred_element_type=jnp.float32)


@functools.cache
def _tc_stage():
    return pl.pallas_call(
        _tc_body,
        grid=(N // R,),
        in_specs=[
            pl.BlockSpec((R, C), lambda i: (i, 0)),
            pl.BlockSpec((C, N), lambda i: (0, 0)),
            pl.BlockSpec((C, C_OUT), lambda i: (0, 0)),
            pl.BlockSpec((C, C_OUT), lambda i: (0, 0)),
            pl.BlockSpec((1, C_OUT), lambda i: (0, 0)),
        ],
        out_specs=[
            pl.BlockSpec((R, K), lambda i: (i, 0)),
            pl.BlockSpec((R, C_OUT), lambda i: (i, 0)),
            pl.BlockSpec((R, C_OUT), lambda i: (i, 0)),
        ],
        out_shape=[
            jax.ShapeDtypeStruct((N, K), jnp.int32),
            jax.ShapeDtypeStruct((N, C_OUT), jnp.float32),
            jax.ShapeDtypeStruct((N, C_OUT), jnp.float32),
        ],
    )


_PTS = 8
_L = 16


@functools.cache
def _sc_gather_max():
    info = plsc.get_sparse_core_info()
    nc, ns = info.num_cores, info.num_subcores
    nw = nc * ns
    per_w = N // nw
    groups = per_w // _PTS
    mesh = plsc.VectorSubcoreMesh(core_axis_name="c", subcore_axis_name="s")

    @functools.partial(
        pl.kernel,
        mesh=mesh,
        out_type=jax.ShapeDtypeStruct((N, C_OUT), jnp.float32),
        scratch_types=[
            pltpu.VMEM((_PTS * K,), jnp.int32),
            pltpu.VMEM((_PTS * K, C_OUT), jnp.float32),
            pltpu.VMEM((_PTS, C_OUT), jnp.float32),
            pltpu.VMEM((_PTS, C_OUT), jnp.float32),
            pltpu.SemaphoreType.DMA,
        ],
    )
    def k(vt_hbm, ut_hbm, nn_hbm, out_hbm, idx_v, rows_v, u_v, o_v, sem):
        wid = lax.axis_index("s") * nc + lax.axis_index("c")
        base = wid * per_w

        def group(g, _):
            p0 = base + g * _PTS
            pltpu.sync_copy(nn_hbm.at[pl.ds(p0 * K, _PTS * K)], idx_v)
            pltpu.async_copy(vt_hbm.at[idx_v], rows_v, sem).wait()
            pltpu.sync_copy(ut_hbm.at[pl.ds(p0, _PTS)], u_v)

            def point(j, _):
                for c in range(C_OUT // _L):
                    sl = pl.ds(c * _L, _L)
                    acc = rows_v[j * K, sl]
                    for r in range(1, K):
                        acc = jnp.maximum(acc, rows_v[j * K + r, sl])
                    o_v[j, sl] = jnp.maximum(acc + u_v[j, sl], 0.0)
                return 0

            lax.fori_loop(0, _PTS, point, 0)
            pltpu.sync_copy(o_v, out_hbm.at[pl.ds(p0, _PTS)])
            return 0

        lax.fori_loop(0, groups, group, 0)

    return k


def kernel(x, W, b):
    xc = x[..., 0]
    xt = jnp.transpose(xc, (0, 2, 1))
    w1 = W[:, :C]
    w2 = W[:, C:]
    wu = jnp.transpose(w1 - w2)
    wv = jnp.transpose(w2)
    b2 = b[None, :]

    tc = _tc_stage()
    sc = _sc_gather_max()
    outs = []
    for bb in range(B):
        nn, ut, vt = tc(xt[bb], xc[bb], wu, wv, b2)
        outs.append(sc(vt, ut, nn.reshape(N * K)))
    g = jnp.stack(outs)
    return jnp.transpose(g, (0, 2, 1))[..., None]


# speedup vs baseline: 34.5543x; 1.0675x over previous
import functools

import jax
import jax.numpy as jnp
from jax import lax
from jax.experimental import pallas as pl
from jax.experimental.pallas import tpu as pltpu
from jax.experimental.pallas import tpu_sc as plsc

B, C, N, K = 4, 256, 4096, 16
C_OUT = 256
R = 256


def _tc_body(xcb_ref, xc_ref, wu_ref, wv_ref, b2_ref, nn_ref, ut_ref, vt_ref):
    row0 = pl.program_id(0) * R
    xcb = xcb_ref[...]
    xca = xc_ref[...]
    s = jax.lax.dot_general(
        xcb, xca, (((0,), (0,)), ((), ())),
        preferred_element_type=jnp.float32)
    colsq = jnp.sum(xca * xca, axis=0, keepdims=True)
    d = colsq - 2.0 * s

    iota = jax.lax.broadcasted_iota(jnp.int32, (R, N), 1)
    inf = jnp.float32(jnp.inf)

    row_ids = row0 + jax.lax.broadcasted_iota(jnp.int32, (R, 1), 0)
    nn_ref[:, pl.ds(0, 1)] = row_ids
    d = jnp.where(iota == row_ids, inf, d)

    nch = N // 128
    Dm = jnp.concatenate(
        [jnp.min(d[:, c * 128:(c + 1) * 128], axis=1, keepdims=True)
         for c in range(nch)], axis=1)
    iota_ch = jax.lax.broadcasted_iota(jnp.int32, (R, nch), 1)
    iota_l = jax.lax.broadcasted_iota(jnp.int32, (R, 128), 1)

    for t in range(1, K):
        m = jnp.min(Dm, axis=1, keepdims=True)
        cstar = jnp.argmin(Dm, axis=1).astype(jnp.int32)[:, None]
        g = d[:, 0:128]
        for c in range(1, nch):
            g = jnp.where(cstar == c, d[:, c * 128:(c + 1) * 128], g)
        g = jnp.where(g < m, inf, g)
        li = jnp.argmin(g, axis=1).astype(jnp.int32)[:, None]
        nn_ref[:, pl.ds(t, 1)] = cstar * 128 + li
        g = jnp.where(iota_l == li, inf, g)
        m2 = jnp.min(g, axis=1, keepdims=True)
        Dm = jnp.where(iota_ch == cstar, m2, Dm)

    ut_ref[...] = jax.lax.dot_general(
        xcb, wu_ref[...], (((0,), (0,)), ((), ())),
        precision=jax.lax.Precision.HIGHEST,
        preferred_element_type=jnp.float32) + b2_ref[...]
    vt_ref[...] = jax.lax.dot_general(
        xcb, wv_ref[...], (((0,), (0,)), ((), ())),
        precision=jax.lax.Precision.HIGHEST,
        preferred_element_type=jnp.float32)


@functools.cache
def _tc_stage():
    return pl.pallas_call(
        _tc_body,
        grid=(N // R,),
        in_specs=[
            pl.BlockSpec((C, R), lambda i: (0, i)),
            pl.BlockSpec((C, N), lambda i: (0, 0)),
            pl.BlockSpec((C, C_OUT), lambda i: (0, 0)),
            pl.BlockSpec((C, C_OUT), lambda i: (0, 0)),
            pl.BlockSpec((1, C_OUT), lambda i: (0, 0)),
        ],
        out_specs=[
            pl.BlockSpec((R, K), lambda i: (i, 0)),
            pl.BlockSpec((R, C_OUT), lambda i: (i, 0)),
            pl.BlockSpec((R, C_OUT), lambda i: (i, 0)),
        ],
        out_shape=[
            jax.ShapeDtypeStruct((N, K), jnp.int32),
            jax.ShapeDtypeStruct((N, C_OUT), jnp.float32),
            jax.ShapeDtypeStruct((N, C_OUT), jnp.float32),
        ],
    )


_PTS = 8
_L = 16


@functools.cache
def _sc_gather_max():
    info = plsc.get_sparse_core_info()
    nc, ns = info.num_cores, info.num_subcores
    nw = nc * ns
    per_w = N // nw
    groups = per_w // _PTS
    mesh = plsc.VectorSubcoreMesh(core_axis_name="c", subcore_axis_name="s")

    @functools.partial(
        pl.kernel,
        mesh=mesh,
        out_type=jax.ShapeDtypeStruct((N, C_OUT), jnp.float32),
        scratch_types=[
            pltpu.VMEM((_PTS * K,), jnp.int32),
            pltpu.VMEM((_PTS * K, C_OUT), jnp.float32),
            pltpu.VMEM((_PTS, C_OUT), jnp.float32),
            pltpu.VMEM((_PTS, C_OUT), jnp.float32),
            pltpu.SemaphoreType.DMA,
        ],
    )
    def k(vt_hbm, ut_hbm, nn_hbm, out_hbm, idx_v, rows_v, u_v, o_v, sem):
        wid = lax.axis_index("s") * nc + lax.axis_index("c")
        base = wid * per_w

        def group(g, _):
            p0 = base + g * _PTS
            pltpu.sync_copy(nn_hbm.at[pl.ds(p0 * K, _PTS * K)], idx_v)
            pltpu.async_copy(vt_hbm.at[idx_v], rows_v, sem).wait()
            pltpu.sync_copy(ut_hbm.at[pl.ds(p0, _PTS)], u_v)

            def point(j, _):
                for c in range(C_OUT // _L):
                    sl = pl.ds(c * _L, _L)
                    acc = rows_v[j * K, sl]
                    for r in range(1, K):
                        acc = jnp.maximum(acc, rows_v[j * K + r, sl])
                    o_v[j, sl] = jnp.maximum(acc + u_v[j, sl], 0.0)
                return 0

            lax.fori_loop(0, _PTS, point, 0)
            pltpu.sync_copy(o_v, out_hbm.at[pl.ds(p0, _PTS)])
            return 0

        lax.fori_loop(0, groups, group, 0)

    return k


def kernel(x, W, b):
    xc = x[..., 0]
    w1 = W[:, :C]
    w2 = W[:, C:]
    wu = jnp.transpose(w1 - w2)
    wv = jnp.transpose(w2)
    b2 = b[None, :]

    tc = _tc_stage()
    sc = _sc_gather_max()
    outs = []
    for bb in range(B):
        nn, ut, vt = tc(xc[bb], xc[bb], wu, wv, b2)
        g = sc(vt, ut, nn.reshape(N * K))
        outs.append(jnp.transpose(g))
    return jnp.stack(outs)[..., None]


# speedup vs baseline: 35.4033x; 1.0246x over previous
import functools

import jax
import jax.numpy as jnp
from jax import lax
from jax.experimental import pallas as pl
from jax.experimental.pallas import tpu as pltpu
from jax.experimental.pallas import tpu_sc as plsc

B, C, N, K = 4, 256, 4096, 16
C_OUT = 256
R = 256


def _tc_body(xcb_ref, xc_ref, wu_ref, wv_ref, b2_ref, nn_ref, ut_ref, vt_ref):
    row0 = pl.program_id(0) * R
    xcb = xcb_ref[...]
    xca = xc_ref[...]
    s = jax.lax.dot_general(
        xcb, xca, (((0,), (0,)), ((), ())),
        preferred_element_type=jnp.float32)
    colsq = jnp.sum(xca * xca, axis=0, keepdims=True)
    d = colsq - 2.0 * s

    iota = jax.lax.broadcasted_iota(jnp.int32, (R, N), 1)
    inf = jnp.float32(jnp.inf)

    row_ids = row0 + jax.lax.broadcasted_iota(jnp.int32, (R, 1), 0)
    nn_ref[:, pl.ds(0, 1)] = row_ids
    d = jnp.where(iota == row_ids, inf, d)

    nch = N // 128
    Dm = jnp.concatenate(
        [jnp.min(d[:, c * 128:(c + 1) * 128], axis=1, keepdims=True)
         for c in range(nch)], axis=1)
    iota_ch = jax.lax.broadcasted_iota(jnp.int32, (R, nch), 1)
    iota_l = jax.lax.broadcasted_iota(jnp.int32, (R, 128), 1)

    for t in range(1, K):
        m = jnp.min(Dm, axis=1, keepdims=True)
        cstar = jnp.argmin(Dm, axis=1).astype(jnp.int32)[:, None]
        g = d[:, 0:128]
        for c in range(1, nch):
            g = jnp.where(cstar == c, d[:, c * 128:(c + 1) * 128], g)
        g = jnp.where(g < m, inf, g)
        li = jnp.argmin(g, axis=1).astype(jnp.int32)[:, None]
        nn_ref[:, pl.ds(t, 1)] = cstar * 128 + li
        g = jnp.where(iota_l == li, inf, g)
        m2 = jnp.min(g, axis=1, keepdims=True)
        Dm = jnp.where(iota_ch == cstar, m2, Dm)

    ut_ref[...] = jax.lax.dot_general(
        xcb, wu_ref[...], (((0,), (0,)), ((), ())),
        precision=jax.lax.Precision.HIGHEST,
        preferred_element_type=jnp.float32) + b2_ref[...]
    vt_ref[...] = jax.lax.dot_general(
        xcb, wv_ref[...], (((0,), (0,)), ((), ())),
        precision=jax.lax.Precision.HIGHEST,
        preferred_element_type=jnp.float32)


@functools.cache
def _tc_stage():
    return pl.pallas_call(
        _tc_body,
        grid=(N // R,),
        in_specs=[
            pl.BlockSpec((C, R), lambda i: (0, i)),
            pl.BlockSpec((C, N), lambda i: (0, 0)),
            pl.BlockSpec((C, C_OUT), lambda i: (0, 0)),
            pl.BlockSpec((C, C_OUT), lambda i: (0, 0)),
            pl.BlockSpec((1, C_OUT), lambda i: (0, 0)),
        ],
        out_specs=[
            pl.BlockSpec((R, K), lambda i: (i, 0)),
            pl.BlockSpec((R, C_OUT), lambda i: (i, 0)),
            pl.BlockSpec((R, C_OUT), lambda i: (i, 0)),
        ],
        out_shape=[
            jax.ShapeDtypeStruct((N, K), jnp.int32),
            jax.ShapeDtypeStruct((N, C_OUT), jnp.float32),
            jax.ShapeDtypeStruct((N, C_OUT), jnp.float32),
        ],
    )


_PTS = 8
_L = 16


@functools.cache
def _sc_gather_max():
    info = plsc.get_sparse_core_info()
    nc, ns = info.num_cores, info.num_subcores
    nw = nc * ns
    per_w = N // nw
    groups = per_w // _PTS
    mesh = plsc.VectorSubcoreMesh(core_axis_name="c", subcore_axis_name="s")

    @functools.partial(
        pl.kernel,
        mesh=mesh,
        out_type=jax.ShapeDtypeStruct((N, C_OUT), jnp.float32),
        scratch_types=[
            pltpu.VMEM((per_w * K,), jnp.int32),
            pltpu.VMEM((2, _PTS * K, C_OUT), jnp.float32),
            pltpu.VMEM((2, _PTS, C_OUT), jnp.float32),
            pltpu.VMEM((2, _PTS, C_OUT), jnp.float32),
            pltpu.SemaphoreType.DMA((2,)),
            pltpu.SemaphoreType.DMA((2,)),
            pltpu.SemaphoreType.DMA((2,)),
        ],
    )
    def k(vt_hbm, ut_hbm, nn_hbm, out_hbm, idx_v, rows_v, u_v, o_v,
          sg, su, so):
        wid = lax.axis_index("s") * nc + lax.axis_index("c")
        base = wid * per_w

        def gather_in(g, par):
            p0 = base + g * _PTS
            return (
                pltpu.make_async_copy(
                    vt_hbm.at[idx_v.at[pl.ds(g * _PTS * K, _PTS * K)]],
                    rows_v.at[par], sg.at[par]),
                pltpu.make_async_copy(
                    ut_hbm.at[pl.ds(p0, _PTS)], u_v.at[par], su.at[par]),
            )

        def store_out(g, par):
            p0 = base + g * _PTS
            return pltpu.make_async_copy(
                o_v.at[par], out_hbm.at[pl.ds(p0, _PTS)], so.at[par])

        pltpu.sync_copy(nn_hbm.at[pl.ds(base * K, per_w * K)], idx_v)
        for d in gather_in(0, 0):
            d.start()

        def group(g, _):
            par = g % 2
            nxt = 1 - par

            @pl.when(g + 1 < groups)
            def _():
                for d in gather_in(g + 1, nxt):
                    d.start()

            for d in gather_in(g, par):
                d.wait()

            @pl.when(g >= 2)
            def _():
                store_out(g - 2, par).wait()

            def point(j, _):
                for c in range(C_OUT // _L):
                    sl = pl.ds(c * _L, _L)
                    acc = rows_v[par, j * K, sl]
                    for r in range(1, K):
                        acc = jnp.maximum(acc, rows_v[par, j * K + r, sl])
                    o_v[par, j, sl] = jnp.maximum(acc + u_v[par, j, sl], 0.0)
                return 0

            lax.fori_loop(0, _PTS, point, 0)
            store_out(g, par).start()
            return 0

        lax.fori_loop(0, groups, group, 0)
        store_out(groups - 2, 0).wait()
        store_out(groups - 1, 1).wait()

    return k


def kernel(x, W, b):
    xc = x[..., 0]
    w1 = W[:, :C]
    w2 = W[:, C:]
    wu = jnp.transpose(w1 - w2)
    wv = jnp.transpose(w2)
    b2 = b[None, :]

    tc = _tc_stage()
    sc = _sc_gather_max()
    outs = []
    for bb in range(B):
        nn, ut, vt = tc(xc[bb], xc[bb], wu, wv, b2)
        g = sc(vt, ut, nn.reshape(N * K))
        outs.append(jnp.transpose(g))
    return jnp.stack(outs)[..., None]
